# baseline (speedup 1.0000x reference)
import functools

import jax
import jax.numpy as jnp
import numpy as np
from jax import lax
from jax.experimental import pallas as pl
from jax.experimental.pallas import tpu as pltpu
from jax.experimental.pallas import tpu_sc as plsc

_L = 32
_B = 32
_V = 100000
_CHUNK = 25600
_NCHUNK = (_V + _CHUNK - 1) // _CHUNK
_TW = 256
_NT = _CHUNK // _TW
_LAST_FULL = (_V - (_NCHUNK - 1) * _CHUNK) // _TW

_NEG_INF = np.float32(-np.inf)
_TINY = np.float32(np.finfo(np.float32).tiny)
_INT_MAX = np.int32(2**31 - 1)

_N_GATHER = _L * _B
_NW = 32
_PER_W = _N_GATHER // _NW


def _sc_gather(logits_flat, flat_idx):
    mesh = plsc.VectorSubcoreMesh(core_axis_name="c", subcore_axis_name="s")

    @functools.partial(
        pl.kernel, mesh=mesh,
        out_type=jax.ShapeDtypeStruct((_N_GATHER,), jnp.float32),
        scratch_types=[
            pltpu.VMEM((_PER_W,), jnp.int32),
            pltpu.VMEM((_PER_W,), jnp.float32),
            pltpu.SemaphoreType.DMA,
        ],
    )
    def k(table_hbm, idx_hbm, out_hbm, idx_v, rows_v, sem):
        wid = lax.axis_index("s") * 2 + lax.axis_index("c")
        base = wid * _PER_W
        pltpu.sync_copy(idx_hbm.at[pl.ds(base, _PER_W)], idx_v)
        pltpu.async_copy(table_hbm.at[idx_v], rows_v, sem).wait()
        pltpu.sync_copy(rows_v, out_hbm.at[pl.ds(base, _PER_W)])

    return k(logits_flat, flat_idx)


def _rotl(x, r):
    return (x << jnp.uint32(r)) | (x >> jnp.uint32(32 - r))


def _threefry_bits(base_j, off_plus_ks1, k0, k1):
    ks0, ks1 = k0, k1
    ks2 = jnp.uint32(0x1BD11BDA) ^ k0 ^ k1
    ks = (ks0, ks1, ks2)
    rots = ((13, 15, 26, 6), (17, 29, 16, 24))
    x1 = base_j + off_plus_ks1
    x0 = x1 + ks0
    first = True
    for i in range(5):
        for r in rots[i % 2]:
            if first:
                first = False
            else:
                x0 = x0 + x1
            x1 = _rotl(x1, r)
            x1 = x0 ^ x1
        x0 = x0 + ks[(i + 1) % 3]
        x1 = x1 + ks[(i + 2) % 3] + jnp.uint32(i + 1)
    return x0 ^ x1


def _neg_gumbel_from_bits(bits):
    fb = (bits >> jnp.uint32(9)) | jnp.uint32(0x3F800000)
    fl = lax.bitcast_convert_type(fb, jnp.float32) - jnp.float32(1.0)
    u = fl + _TINY
    return jnp.log(-jnp.log(u))


def _sample_kernel(keys_ref, inp_ref, msk_ref, ginp_ref, logits_ref,
                   gen_ref, logp_ref, bv_ref, bt_ref, sa_ref):
    t = pl.program_id(0)
    c = pl.program_id(1)

    @pl.when(c == 0)
    def _init():
        bv_ref[...] = jnp.full((_B, _TW), _NEG_INF, jnp.float32)
        bt_ref[...] = jnp.zeros((_B, _TW), jnp.int32)
        sa_ref[...] = jnp.zeros((_B, _TW), jnp.float32)

    k0 = keys_ref[2 * t].astype(jnp.uint32)
    k1 = keys_ref[2 * t + 1].astype(jnp.uint32)

    lane = lax.broadcasted_iota(jnp.int32, (_B, _TW), 1)
    row = lax.broadcasted_iota(jnp.uint32, (_B, _TW), 0)
    base_j = row * jnp.uint32(_V) + lane.astype(jnp.uint32)

    last = c == _NCHUNK - 1
    npairs = jnp.where(last, _LAST_FULL // 2, _NT // 2)

    def tile(tidx, carry, masked):
        bv, bt, sa = carry
        off = tidx * _TW
        lo = logits_ref[0, :, pl.ds(off, _TW)]
        goff = c * _CHUNK + off
        ng = _neg_gumbel_from_bits(
            _threefry_bits(base_j, goff.astype(jnp.uint32) + k1, k0, k1))
        gt = c * _NT + tidx
        if masked:
            valid = lane + goff < _V
            val = jnp.where(valid, lo - ng, _NEG_INF)
            sa = sa + jnp.where(valid, jnp.exp(lo), jnp.float32(0.0))
        else:
            val = lo - ng
            sa = sa + jnp.exp(lo)
        upd = val > bv
        bv = jnp.maximum(bv, val)
        bt = jnp.where(upd, gt, bt)
        return bv, bt, sa

    def pair_body(i, carry):
        carry = tile(2 * i, carry, masked=False)
        carry = tile(2 * i + 1, carry, masked=False)
        return carry

    def tail_body(_, carry):
        return tile(_LAST_FULL, carry, masked=True)

    carry = (bv_ref[...], bt_ref[...], sa_ref[...])
    carry = lax.fori_loop(0, npairs, pair_body, carry)
    carry = lax.fori_loop(0, jnp.where(last, 1, 0), tail_body, carry)
    bv_ref[...], bt_ref[...], sa_ref[...] = carry

    @pl.when(last)
    def _finalize():
        bv, bt, sa = bv_ref[...], bt_ref[...], sa_ref[...]
        col = bt * _TW + lane
        cmax = jnp.max(bv, axis=1, keepdims=True)
        eq = bv == cmax
        samp = jnp.min(jnp.where(eq, col, _INT_MAX), axis=1, keepdims=True)
        csum = jnp.sum(sa, axis=1, keepdims=True)

        row1 = lax.broadcasted_iota(jnp.uint32, (_B, 1), 0)
        ngb = _neg_gumbel_from_bits(
            _threefry_bits(row1 * jnp.uint32(_V), samp.astype(jnp.uint32) + k1,
                           k0, k1))
        logit_b = cmax + ngb

        inp = inp_ref[0]
        msk = msk_ref[0] != 0
        gen_ref[0] = jnp.where(msk, samp, inp)
        logp_ref[0] = jnp.where(msk, logit_b, ginp_ref[0]) - jnp.log(csum)


@jax.jit
def _run(keys_flat, inp_lb, msk_lb, ginp_lb, gen_logits):
    grid_spec = pltpu.PrefetchScalarGridSpec(
        num_scalar_prefetch=1,
        grid=(_L, _NCHUNK),
        in_specs=[
            pl.BlockSpec((1, _B, 1), lambda t, c, keys: (t, 0, 0)),
            pl.BlockSpec((1, _B, 1), lambda t, c, keys: (t, 0, 0)),
            pl.BlockSpec((1, _B, 1), lambda t, c, keys: (t, 0, 0)),
            pl.BlockSpec((1, _B, _CHUNK), lambda t, c, keys: (t, 0, c)),
        ],
        out_specs=[
            pl.BlockSpec((1, _B, 1), lambda t, c, keys: (t, 0, 0)),
            pl.BlockSpec((1, _B, 1), lambda t, c, keys: (t, 0, 0)),
        ],
        scratch_shapes=[
            pltpu.VMEM((_B, _TW), jnp.float32),
            pltpu.VMEM((_B, _TW), jnp.int32),
            pltpu.VMEM((_B, _TW), jnp.float32),
        ],
    )
    gen, logp = pl.pallas_call(
        _sample_kernel,
        grid_spec=grid_spec,
        out_shape=[
            jax.ShapeDtypeStruct((_L, _B, 1), jnp.int32),
            jax.ShapeDtypeStruct((_L, _B, 1), jnp.float32),
        ],
        compiler_params=pltpu.CompilerParams(
            dimension_semantics=("arbitrary", "arbitrary"),
        ),
    )(keys_flat, inp_lb, msk_lb, ginp_lb, gen_logits)
    return gen, logp


def kernel(input_tensor, mask_tensor, gen_logits):
    L, B, V = gen_logits.shape
    assert (L, B, V) == (_L, _B, _V)

    sample_key = jax.random.key(42)
    keys = jax.vmap(
        lambda t: jax.random.key_data(jax.random.fold_in(sample_key, t))
    )(jnp.arange(L, dtype=jnp.uint32))
    keys_flat = keys.reshape(-1).astype(jnp.int32)

    inp_lb = input_tensor.T.reshape(L, B, 1)
    msk_lb = mask_tensor.T.reshape(L, B, 1)

    t_idx = jnp.arange(L, dtype=jnp.int32)[:, None]
    flat_idx = (t_idx * (B * V) + jnp.arange(B, dtype=jnp.int32)[None, :] * V
                + input_tensor.T).reshape(-1)
    ginp = _sc_gather(gen_logits.reshape(-1), flat_idx)
    ginp_lb = ginp.reshape(L, B, 1)

    gen, logp = _run(keys_flat, inp_lb, msk_lb, ginp_lb, gen_logits)
    generated = gen.reshape(L, B).T
    log_probs = logp.reshape(L, B).T
    return generated, log_probs


# speedup vs baseline: 1.2896x; 1.2896x over previous
import functools

import jax
import jax.numpy as jnp
import numpy as np
from jax import lax
from jax.experimental import pallas as pl
from jax.experimental.pallas import tpu as pltpu

_L = 32
_B = 32
_V = 100000
_CHUNK = 25600
_NCHUNK = (_V + _CHUNK - 1) // _CHUNK
_TW = 256
_NT = _CHUNK // _TW
_LAST_FULL = (_V - (_NCHUNK - 1) * _CHUNK) // _TW

_NEG_INF = np.float32(-np.inf)
_TINY = np.float32(np.finfo(np.float32).tiny)
_INT_MAX = np.int32(2**31 - 1)


def _rotl(x, r):
    return (x << jnp.uint32(r)) | (x >> jnp.uint32(32 - r))


def _threefry_bits(base_j, off_plus_ks1, k0, k1):
    ks0, ks1 = k0, k1
    ks2 = jnp.uint32(0x1BD11BDA) ^ k0 ^ k1
    ks = (ks0, ks1, ks2)
    rots = ((13, 15, 26, 6), (17, 29, 16, 24))
    x1 = base_j + off_plus_ks1
    x0 = x1 + ks0
    first = True
    for i in range(5):
        for r in rots[i % 2]:
            if first:
                first = False
            else:
                x0 = x0 + x1
            x1 = _rotl(x1, r)
            x1 = x0 ^ x1
        x0 = x0 + ks[(i + 1) % 3]
        x1 = x1 + ks[(i + 2) % 3] + jnp.uint32(i + 1)
    return x0 ^ x1


def _neg_gumbel_from_bits(bits):
    fb = (bits >> jnp.uint32(9)) | jnp.uint32(0x3F800000)
    fl = lax.bitcast_convert_type(fb, jnp.float32) - jnp.float32(1.0)
    u = fl + _TINY
    return jnp.log(-jnp.log(u))


def _sample_kernel(keys_ref, inp_ref, msk_ref, logits_ref, gen_ref, logp_ref,
                   bv_ref, bc_ref, sa_ref, ia_ref):
    t = pl.program_id(0)
    c = pl.program_id(1)

    @pl.when(c == 0)
    def _init():
        bv_ref[...] = jnp.full((_B, _TW), _NEG_INF, jnp.float32)
        bc_ref[...] = jnp.zeros((_B, _TW), jnp.int32)
        sa_ref[...] = jnp.zeros((_B, _TW), jnp.float32)
        ia_ref[...] = jnp.full((_B, _TW), _NEG_INF, jnp.float32)

    k0 = keys_ref[2 * t].astype(jnp.uint32)
    k1 = keys_ref[2 * t + 1].astype(jnp.uint32)
    inp = inp_ref[0]

    lane = lax.broadcasted_iota(jnp.int32, (_B, _TW), 1)
    row = lax.broadcasted_iota(jnp.uint32, (_B, _TW), 0)
    base_j = row * jnp.uint32(_V) + lane.astype(jnp.uint32)

    last = c == _NCHUNK - 1
    npairs = jnp.where(last, _LAST_FULL // 2, _NT // 2)

    def tile(off, carry, masked):
        bv, bc, sa, ia = carry
        lo = logits_ref[0, :, pl.ds(off, _TW)]
        goff = c * _CHUNK + off
        ng = _neg_gumbel_from_bits(
            _threefry_bits(base_j, goff.astype(jnp.uint32) + k1, k0, k1))
        col = lane + goff
        if masked:
            valid = col < _V
            val = jnp.where(valid, lo - ng, _NEG_INF)
            sa = sa + jnp.where(valid, jnp.exp(lo), jnp.float32(0.0))
        else:
            val = lo - ng
            sa = sa + jnp.exp(lo)
        upd = val > bv
        bv = jnp.maximum(bv, val)
        bc = jnp.where(upd, col, bc)
        ia = jnp.maximum(ia, jnp.where(col == inp, lo, _NEG_INF))
        return bv, bc, sa, ia

    def pair_body(i, carry):
        carry = tile((2 * i) * _TW, carry, masked=False)
        carry = tile((2 * i + 1) * _TW, carry, masked=False)
        return carry

    def tail_body(_, carry):
        return tile(_LAST_FULL * _TW, carry, masked=True)

    carry = (bv_ref[...], bc_ref[...], sa_ref[...], ia_ref[...])
    carry = lax.fori_loop(0, npairs, pair_body, carry)
    carry = lax.fori_loop(0, jnp.where(last, 1, 0), tail_body, carry)
    bv_ref[...], bc_ref[...], sa_ref[...], ia_ref[...] = carry

    @pl.when(last)
    def _finalize():
        bv, bc, sa, ia = bv_ref[...], bc_ref[...], sa_ref[...], ia_ref[...]
        cmax = jnp.max(bv, axis=1, keepdims=True)
        eq = bv == cmax
        samp = jnp.min(jnp.where(eq, bc, _INT_MAX), axis=1, keepdims=True)
        csum = jnp.sum(sa, axis=1, keepdims=True)
        cinp = jnp.max(ia, axis=1, keepdims=True)

        row1 = lax.broadcasted_iota(jnp.uint32, (_B, 1), 0)
        ngb = _neg_gumbel_from_bits(
            _threefry_bits(row1 * jnp.uint32(_V), samp.astype(jnp.uint32) + k1,
                           k0, k1))
        logit_b = cmax + ngb

        msk = msk_ref[0] != 0
        gen_ref[0] = jnp.where(msk, samp, inp)
        logp_ref[0] = jnp.where(msk, logit_b, cinp) - jnp.log(csum)


@jax.jit
def _run(keys_flat, inp_lb, msk_lb, gen_logits):
    grid_spec = pltpu.PrefetchScalarGridSpec(
        num_scalar_prefetch=1,
        grid=(_L, _NCHUNK),
        in_specs=[
            pl.BlockSpec((1, _B, 1), lambda t, c, keys: (t, 0, 0)),
            pl.BlockSpec((1, _B, 1), lambda t, c, keys: (t, 0, 0)),
            pl.BlockSpec((1, _B, _CHUNK), lambda t, c, keys: (t, 0, c)),
        ],
        out_specs=[
            pl.BlockSpec((1, _B, 1), lambda t, c, keys: (t, 0, 0)),
            pl.BlockSpec((1, _B, 1), lambda t, c, keys: (t, 0, 0)),
        ],
        scratch_shapes=[
            pltpu.VMEM((_B, _TW), jnp.float32),
            pltpu.VMEM((_B, _TW), jnp.int32),
            pltpu.VMEM((_B, _TW), jnp.float32),
            pltpu.VMEM((_B, _TW), jnp.float32),
        ],
    )
    gen, logp = pl.pallas_call(
        _sample_kernel,
        grid_spec=grid_spec,
        out_shape=[
            jax.ShapeDtypeStruct((_L, _B, 1), jnp.int32),
            jax.ShapeDtypeStruct((_L, _B, 1), jnp.float32),
        ],
        compiler_params=pltpu.CompilerParams(
            dimension_semantics=("arbitrary", "arbitrary"),
        ),
    )(keys_flat, inp_lb, msk_lb, gen_logits)
    return gen, logp


def kernel(input_tensor, mask_tensor, gen_logits):
    L, B, V = gen_logits.shape
    assert (L, B, V) == (_L, _B, _V)

    sample_key = jax.random.key(42)
    keys = jax.vmap(
        lambda t: jax.random.key_data(jax.random.fold_in(sample_key, t))
    )(jnp.arange(L, dtype=jnp.uint32))
    keys_flat = keys.reshape(-1).astype(jnp.int32)

    inp_lb = input_tensor.T.reshape(L, B, 1)
    msk_lb = mask_tensor.T.reshape(L, B, 1)

    gen, logp = _run(keys_flat, inp_lb, msk_lb, gen_logits)
    generated = gen.reshape(L, B).T
    log_probs = logp.reshape(L, B).T
    return generated, log_probs


# speedup vs baseline: 1.3067x; 1.0132x over previous
import functools

import jax
import jax.numpy as jnp
import numpy as np
from jax import lax
from jax.experimental import pallas as pl
from jax.experimental.pallas import tpu as pltpu

_L = 32
_B = 32
_V = 100000
_CHUNK = 25600
_NCHUNK = (_V + _CHUNK - 1) // _CHUNK
_TW = 256
_NT = _CHUNK // _TW
_LAST_FULL = (_V - (_NCHUNK - 1) * _CHUNK) // _TW

_NEG_INF = np.float32(-np.inf)
_TINY = np.float32(np.finfo(np.float32).tiny)
_INT_MAX = np.int32(2**31 - 1)


def _rotl(x, r):
    return (x << jnp.uint32(r)) | (x >> jnp.uint32(32 - r))


def _threefry_bits(base_j, off_plus_ks1, k0, k1):
    ks0, ks1 = k0, k1
    ks2 = jnp.uint32(0x1BD11BDA) ^ k0 ^ k1
    ks = (ks0, ks1, ks2)
    rots = ((13, 15, 26, 6), (17, 29, 16, 24))
    x1 = base_j + off_plus_ks1
    x0 = x1 + ks0
    first = True
    for i in range(5):
        for r in rots[i % 2]:
            if first:
                first = False
            else:
                x0 = x0 + x1
            x1 = _rotl(x1, r)
            x1 = x0 ^ x1
        x0 = x0 + ks[(i + 1) % 3]
        x1 = x1 + ks[(i + 2) % 3] + jnp.uint32(i + 1)
    return x0 ^ x1


def _neg_gumbel_from_bits(bits):
    fb = (bits >> jnp.uint32(9)) | jnp.uint32(0x3F800000)
    fl = lax.bitcast_convert_type(fb, jnp.float32) - jnp.float32(1.0)
    u = fl + _TINY
    return jnp.log(-jnp.log(u))


def _sample_kernel(keys_ref, inp_ref, msk_ref, logits_ref, gen_ref, logp_ref,
                   bv_ref, bc_ref, sa_ref, ia_ref):
    t = pl.program_id(0)
    c = pl.program_id(1)

    @pl.when(c == 0)
    def _init():
        bv_ref[...] = jnp.full((_B, _TW), _NEG_INF, jnp.float32)
        bc_ref[...] = jnp.zeros((_B, _TW), jnp.int32)
        sa_ref[...] = jnp.zeros((_B, _TW), jnp.float32)
        ia_ref[...] = jnp.full((_B, _TW), _NEG_INF, jnp.float32)

    k0 = keys_ref[2 * t].astype(jnp.uint32)
    k1 = keys_ref[2 * t + 1].astype(jnp.uint32)
    inp = inp_ref[0]

    lane = lax.broadcasted_iota(jnp.int32, (_B, _TW), 1)
    row = lax.broadcasted_iota(jnp.uint32, (_B, _TW), 0)
    base_j = row * jnp.uint32(_V) + lane.astype(jnp.uint32)

    last = c == _NCHUNK - 1
    ntriples = jnp.where(last, _LAST_FULL // 3, _NT // 3)
    nsingle = jnp.where(last, 0, 1)

    def tile(off, carry, masked):
        bv, bc, sa, ia = carry
        lo = logits_ref[0, :, pl.ds(off, _TW)]
        goff = c * _CHUNK + off
        ng = _neg_gumbel_from_bits(
            _threefry_bits(base_j, goff.astype(jnp.uint32) + k1, k0, k1))
        col = lane + goff
        if masked:
            valid = col < _V
            val = jnp.where(valid, lo - ng, _NEG_INF)
            sa = sa + jnp.where(valid, jnp.exp(lo), jnp.float32(0.0))
        else:
            val = lo - ng
            sa = sa + jnp.exp(lo)
        upd = val > bv
        bv = jnp.maximum(bv, val)
        bc = jnp.where(upd, col, bc)
        ia = jnp.maximum(ia, jnp.where(col == inp, lo, _NEG_INF))
        return bv, bc, sa, ia

    def triple_body(i, carry):
        carry = tile((3 * i) * _TW, carry, masked=False)
        carry = tile((3 * i + 1) * _TW, carry, masked=False)
        carry = tile((3 * i + 2) * _TW, carry, masked=False)
        return carry

    def single_body(_, carry):
        return tile((_NT - 1) * _TW, carry, masked=False)

    def tail_body(_, carry):
        return tile(_LAST_FULL * _TW, carry, masked=True)

    carry = (bv_ref[...], bc_ref[...], sa_ref[...], ia_ref[...])
    carry = lax.fori_loop(0, ntriples, triple_body, carry)
    carry = lax.fori_loop(0, nsingle, single_body, carry)
    carry = lax.fori_loop(0, jnp.where(last, 1, 0), tail_body, carry)
    bv_ref[...], bc_ref[...], sa_ref[...], ia_ref[...] = carry

    @pl.when(last)
    def _finalize():
        bv, bc, sa, ia = bv_ref[...], bc_ref[...], sa_ref[...], ia_ref[...]
        cmax = jnp.max(bv, axis=1, keepdims=True)
        eq = bv == cmax
        samp = jnp.min(jnp.where(eq, bc, _INT_MAX), axis=1, keepdims=True)
        csum = jnp.sum(sa, axis=1, keepdims=True)
        cinp = jnp.max(ia, axis=1, keepdims=True)

        row1 = lax.broadcasted_iota(jnp.uint32, (_B, 1), 0)
        ngb = _neg_gumbel_from_bits(
            _threefry_bits(row1 * jnp.uint32(_V), samp.astype(jnp.uint32) + k1,
                           k0, k1))
        logit_b = cmax + ngb

        msk = msk_ref[0] != 0
        gen_ref[0] = jnp.where(msk, samp, inp)
        logp_ref[0] = jnp.where(msk, logit_b, cinp) - jnp.log(csum)


@jax.jit
def _run(keys_flat, inp_lb, msk_lb, gen_logits):
    grid_spec = pltpu.PrefetchScalarGridSpec(
        num_scalar_prefetch=1,
        grid=(_L, _NCHUNK),
        in_specs=[
            pl.BlockSpec((1, _B, 1), lambda t, c, keys: (t, 0, 0)),
            pl.BlockSpec((1, _B, 1), lambda t, c, keys: (t, 0, 0)),
            pl.BlockSpec((1, _B, _CHUNK), lambda t, c, keys: (t, 0, c)),
        ],
        out_specs=[
            pl.BlockSpec((1, _B, 1), lambda t, c, keys: (t, 0, 0)),
            pl.BlockSpec((1, _B, 1), lambda t, c, keys: (t, 0, 0)),
        ],
        scratch_shapes=[
            pltpu.VMEM((_B, _TW), jnp.float32),
            pltpu.VMEM((_B, _TW), jnp.int32),
            pltpu.VMEM((_B, _TW), jnp.float32),
            pltpu.VMEM((_B, _TW), jnp.float32),
        ],
    )
    gen, logp = pl.pallas_call(
        _sample_kernel,
        grid_spec=grid_spec,
        out_shape=[
            jax.ShapeDtypeStruct((_L, _B, 1), jnp.int32),
            jax.ShapeDtypeStruct((_L, _B, 1), jnp.float32),
        ],
        compiler_params=pltpu.CompilerParams(
            dimension_semantics=("arbitrary", "arbitrary"),
        ),
    )(keys_flat, inp_lb, msk_lb, gen_logits)
    return gen, logp


def kernel(input_tensor, mask_tensor, gen_logits):
    L, B, V = gen_logits.shape
    assert (L, B, V) == (_L, _B, _V)

    sample_key = jax.random.key(42)
    keys = jax.vmap(
        lambda t: jax.random.key_data(jax.random.fold_in(sample_key, t))
    )(jnp.arange(L, dtype=jnp.uint32))
    keys_flat = keys.reshape(-1).astype(jnp.int32)

    inp_lb = input_tensor.T.reshape(L, B, 1)
    msk_lb = mask_tensor.T.reshape(L, B, 1)

    gen, logp = _run(keys_flat, inp_lb, msk_lb, gen_logits)
    generated = gen.reshape(L, B).T
    log_probs = logp.reshape(L, B).T
    return generated, log_probs


# speedup vs baseline: 1.3195x; 1.0098x over previous
import functools

import jax
import jax.numpy as jnp
import numpy as np
from jax import lax
from jax.experimental import pallas as pl
from jax.experimental.pallas import tpu as pltpu

_L = 32
_B = 32
_V = 100000
_CHUNK = 25600
_NCHUNK = (_V + _CHUNK - 1) // _CHUNK
_TW = 256
_NT = _CHUNK // _TW
_LAST_FULL = (_V - (_NCHUNK - 1) * _CHUNK) // _TW

_NEG_INF = np.float32(-np.inf)
_TINY = np.float32(np.finfo(np.float32).tiny)
_INT_MAX = np.int32(2**31 - 1)


def _rotl(x, r):
    return (x << jnp.uint32(r)) | (x >> jnp.uint32(32 - r))


def _threefry_bits(base_j, off_plus_ks1, k0, k1):
    ks0, ks1 = k0, k1
    ks2 = jnp.uint32(0x1BD11BDA) ^ k0 ^ k1
    ks = (ks0, ks1, ks2)
    rots = ((13, 15, 26, 6), (17, 29, 16, 24))
    x1 = base_j + off_plus_ks1
    x0 = x1 + ks0
    first = True
    for i in range(5):
        for r in rots[i % 2]:
            if first:
                first = False
            else:
                x0 = x0 + x1
            x1 = _rotl(x1, r)
            x1 = x0 ^ x1
        x0 = x0 + ks[(i + 1) % 3]
        x1 = x1 + ks[(i + 2) % 3] + jnp.uint32(i + 1)
    return x0 ^ x1


def _neg_gumbel_from_bits(bits):
    fb = (bits >> jnp.uint32(9)) | jnp.uint32(0x3F800000)
    fl = lax.bitcast_convert_type(fb, jnp.float32) - jnp.float32(1.0)
    u = fl + _TINY
    return jnp.log(-jnp.log(u))


def _sample_kernel(keys_ref, inp_ref, msk_ref, logits_ref, gen_ref, logp_ref,
                   bv_ref, bc_ref, sa_ref, ia_ref):
    t = pl.program_id(0)
    c = pl.program_id(1)

    @pl.when(c == 0)
    def _init():
        bv_ref[...] = jnp.full((_B, _TW), _NEG_INF, jnp.float32)
        bc_ref[...] = jnp.zeros((_B, _TW), jnp.int32)
        sa_ref[...] = jnp.zeros((_B, _TW), jnp.float32)
        ia_ref[...] = jnp.full((_B, _TW), _NEG_INF, jnp.float32)

    k0 = keys_ref[2 * t].astype(jnp.uint32)
    k1 = keys_ref[2 * t + 1].astype(jnp.uint32)
    inp = inp_ref[0]

    lane = lax.broadcasted_iota(jnp.int32, (_B, _TW), 1)
    row = lax.broadcasted_iota(jnp.uint32, (_B, _TW), 0)
    base_j = row * jnp.uint32(_V) + lane.astype(jnp.uint32)

    last = c == _NCHUNK - 1
    nquads = jnp.where(last, _LAST_FULL // 4, _NT // 4)
    npair = jnp.where(last, 1, 0)

    def tile(off, carry, masked):
        bv, bc, sa, ia = carry
        lo = logits_ref[0, :, pl.ds(off, _TW)]
        goff = c * _CHUNK + off
        ng = _neg_gumbel_from_bits(
            _threefry_bits(base_j, goff.astype(jnp.uint32) + k1, k0, k1))
        col = lane + goff
        if masked:
            valid = col < _V
            val = jnp.where(valid, lo - ng, _NEG_INF)
            sa = sa + jnp.where(valid, jnp.exp(lo), jnp.float32(0.0))
        else:
            val = lo - ng
            sa = sa + jnp.exp(lo)
        upd = val > bv
        bv = jnp.maximum(bv, val)
        bc = jnp.where(upd, col, bc)
        ia = jnp.maximum(ia, jnp.where(col == inp, lo, _NEG_INF))
        return bv, bc, sa, ia

    def quad_body(i, carry):
        carry = tile((4 * i) * _TW, carry, masked=False)
        carry = tile((4 * i + 1) * _TW, carry, masked=False)
        carry = tile((4 * i + 2) * _TW, carry, masked=False)
        carry = tile((4 * i + 3) * _TW, carry, masked=False)
        return carry

    def pair_body(_, carry):
        carry = tile((_LAST_FULL - 2) * _TW, carry, masked=False)
        carry = tile((_LAST_FULL - 1) * _TW, carry, masked=False)
        return carry

    def tail_body(_, carry):
        return tile(_LAST_FULL * _TW, carry, masked=True)

    carry = (bv_ref[...], bc_ref[...], sa_ref[...], ia_ref[...])
    carry = lax.fori_loop(0, nquads, quad_body, carry)
    carry = lax.fori_loop(0, npair, pair_body, carry)
    carry = lax.fori_loop(0, jnp.where(last, 1, 0), tail_body, carry)
    bv_ref[...], bc_ref[...], sa_ref[...], ia_ref[...] = carry

    @pl.when(last)
    def _finalize():
        bv, bc, sa, ia = bv_ref[...], bc_ref[...], sa_ref[...], ia_ref[...]
        cmax = jnp.max(bv, axis=1, keepdims=True)
        eq = bv == cmax
        samp = jnp.min(jnp.where(eq, bc, _INT_MAX), axis=1, keepdims=True)
        csum = jnp.sum(sa, axis=1, keepdims=True)
        cinp = jnp.max(ia, axis=1, keepdims=True)

        row1 = lax.broadcasted_iota(jnp.uint32, (_B, 1), 0)
        ngb = _neg_gumbel_from_bits(
            _threefry_bits(row1 * jnp.uint32(_V), samp.astype(jnp.uint32) + k1,
                           k0, k1))
        logit_b = cmax + ngb

        msk = msk_ref[0] != 0
        gen_ref[0] = jnp.where(msk, samp, inp)
        logp_ref[0] = jnp.where(msk, logit_b, cinp) - jnp.log(csum)


@jax.jit
def _run(keys_flat, inp_lb, msk_lb, gen_logits):
    grid_spec = pltpu.PrefetchScalarGridSpec(
        num_scalar_prefetch=1,
        grid=(_L, _NCHUNK),
        in_specs=[
            pl.BlockSpec((1, _B, 1), lambda t, c, keys: (t, 0, 0)),
            pl.BlockSpec((1, _B, 1), lambda t, c, keys: (t, 0, 0)),
            pl.BlockSpec((1, _B, _CHUNK), lambda t, c, keys: (t, 0, c)),
        ],
        out_specs=[
            pl.BlockSpec((1, _B, 1), lambda t, c, keys: (t, 0, 0)),
            pl.BlockSpec((1, _B, 1), lambda t, c, keys: (t, 0, 0)),
        ],
        scratch_shapes=[
            pltpu.VMEM((_B, _TW), jnp.float32),
            pltpu.VMEM((_B, _TW), jnp.int32),
            pltpu.VMEM((_B, _TW), jnp.float32),
            pltpu.VMEM((_B, _TW), jnp.float32),
        ],
    )
    gen, logp = pl.pallas_call(
        _sample_kernel,
        grid_spec=grid_spec,
        out_shape=[
            jax.ShapeDtypeStruct((_L, _B, 1), jnp.int32),
            jax.ShapeDtypeStruct((_L, _B, 1), jnp.float32),
        ],
        compiler_params=pltpu.CompilerParams(
            dimension_semantics=("arbitrary", "arbitrary"),
        ),
    )(keys_flat, inp_lb, msk_lb, gen_logits)
    return gen, logp


def kernel(input_tensor, mask_tensor, gen_logits):
    L, B, V = gen_logits.shape
    assert (L, B, V) == (_L, _B, _V)

    sample_key = jax.random.key(42)
    keys = jax.vmap(
        lambda t: jax.random.key_data(jax.random.fold_in(sample_key, t))
    )(jnp.arange(L, dtype=jnp.uint32))
    keys_flat = keys.reshape(-1).astype(jnp.int32)

    inp_lb = input_tensor.T.reshape(L, B, 1)
    msk_lb = mask_tensor.T.reshape(L, B, 1)

    gen, logp = _run(keys_flat, inp_lb, msk_lb, gen_logits)
    generated = gen.reshape(L, B).T
    log_probs = logp.reshape(L, B).T
    return generated, log_probs


# speedup vs baseline: 1.3278x; 1.0063x over previous
import functools

import jax
import jax.numpy as jnp
import numpy as np
from jax import lax
from jax.experimental import pallas as pl
from jax.experimental.pallas import tpu as pltpu

_L = 32
_B = 32
_V = 100000
_CHUNK = 25600
_NCHUNK = (_V + _CHUNK - 1) // _CHUNK
_TW = 256
_NT = _CHUNK // _TW
_LAST_FULL = (_V - (_NCHUNK - 1) * _CHUNK) // _TW

_NEG_INF = np.float32(-np.inf)
_TINY = np.float32(np.finfo(np.float32).tiny)
_INT_MAX = np.int32(2**31 - 1)


def _rotl(x, r):
    return (x << jnp.uint32(r)) | (x >> jnp.uint32(32 - r))


def _threefry_bits(base_j, off_plus_ks1, k0, k1):
    ks0, ks1 = k0, k1
    ks2 = jnp.uint32(0x1BD11BDA) ^ k0 ^ k1
    ks = (ks0, ks1, ks2)
    rots = ((13, 15, 26, 6), (17, 29, 16, 24))
    x1 = base_j + off_plus_ks1
    x0 = x1 + ks0
    first = True
    for i in range(5):
        for r in rots[i % 2]:
            if first:
                first = False
            else:
                x0 = x0 + x1
            x1 = _rotl(x1, r)
            x1 = x0 ^ x1
        x0 = x0 + ks[(i + 1) % 3]
        x1 = x1 + ks[(i + 2) % 3] + jnp.uint32(i + 1)
    return x0 ^ x1


def _neg_gumbel_from_bits(bits):
    fb = (bits >> jnp.uint32(9)) | jnp.uint32(0x3F800000)
    fl = lax.bitcast_convert_type(fb, jnp.float32) - jnp.float32(1.0)
    u = fl + _TINY
    return jnp.log(-jnp.log(u))


def _sample_kernel(keys_ref, inp_ref, msk_ref, logits_ref, gen_ref, logp_ref,
                   bv_ref, bc_ref, sa_ref, ia_ref):
    t = pl.program_id(0)
    c = pl.program_id(1)

    @pl.when(c == 0)
    def _init():
        bv_ref[...] = jnp.full((_B, _TW), _NEG_INF, jnp.float32)
        bc_ref[...] = jnp.zeros((_B, _TW), jnp.int32)
        sa_ref[...] = jnp.zeros((_B, _TW), jnp.float32)
        ia_ref[...] = jnp.full((_B, _TW), _NEG_INF, jnp.float32)

    k0 = keys_ref[2 * t].astype(jnp.uint32)
    k1 = keys_ref[2 * t + 1].astype(jnp.uint32)
    inp = inp_ref[0]

    lane = lax.broadcasted_iota(jnp.int32, (_B, _TW), 1)
    row = lax.broadcasted_iota(jnp.uint32, (_B, _TW), 0)
    base_j = row * jnp.uint32(_V) + lane.astype(jnp.uint32)

    last = c == _NCHUNK - 1
    ngroups = jnp.where(last, _LAST_FULL // 5, _NT // 5)

    def tile(off, carry, masked):
        bv, bc, sa, ia = carry
        lo = logits_ref[0, :, pl.ds(off, _TW)]
        goff = c * _CHUNK + off
        ng = _neg_gumbel_from_bits(
            _threefry_bits(base_j, goff.astype(jnp.uint32) + k1, k0, k1))
        col = lane + goff
        if masked:
            valid = col < _V
            val = jnp.where(valid, lo - ng, _NEG_INF)
            sa = sa + jnp.where(valid, jnp.exp(lo), jnp.float32(0.0))
        else:
            val = lo - ng
            sa = sa + jnp.exp(lo)
        upd = val > bv
        bv = jnp.maximum(bv, val)
        bc = jnp.where(upd, col, bc)
        ia = jnp.maximum(ia, jnp.where(col == inp, lo, _NEG_INF))
        return bv, bc, sa, ia

    def group_body(i, carry):
        for u in range(5):
            carry = tile((5 * i + u) * _TW, carry, masked=False)
        return carry

    def tail_body(_, carry):
        return tile(_LAST_FULL * _TW, carry, masked=True)

    carry = (bv_ref[...], bc_ref[...], sa_ref[...], ia_ref[...])
    carry = lax.fori_loop(0, ngroups, group_body, carry)
    carry = lax.fori_loop(0, jnp.where(last, 1, 0), tail_body, carry)
    bv_ref[...], bc_ref[...], sa_ref[...], ia_ref[...] = carry

    @pl.when(last)
    def _finalize():
        bv, bc, sa, ia = bv_ref[...], bc_ref[...], sa_ref[...], ia_ref[...]
        cmax = jnp.max(bv, axis=1, keepdims=True)
        eq = bv == cmax
        samp = jnp.min(jnp.where(eq, bc, _INT_MAX), axis=1, keepdims=True)
        csum = jnp.sum(sa, axis=1, keepdims=True)
        cinp = jnp.max(ia, axis=1, keepdims=True)

        row1 = lax.broadcasted_iota(jnp.uint32, (_B, 1), 0)
        ngb = _neg_gumbel_from_bits(
            _threefry_bits(row1 * jnp.uint32(_V), samp.astype(jnp.uint32) + k1,
                           k0, k1))
        logit_b = cmax + ngb

        msk = msk_ref[0] != 0
        gen_ref[0] = jnp.where(msk, samp, inp)
        logp_ref[0] = jnp.where(msk, logit_b, cinp) - jnp.log(csum)


@jax.jit
def _run(keys_flat, inp_lb, msk_lb, gen_logits):
    grid_spec = pltpu.PrefetchScalarGridSpec(
        num_scalar_prefetch=1,
        grid=(_L, _NCHUNK),
        in_specs=[
            pl.BlockSpec((1, _B, 1), lambda t, c, keys: (t, 0, 0)),
            pl.BlockSpec((1, _B, 1), lambda t, c, keys: (t, 0, 0)),
            pl.BlockSpec((1, _B, _CHUNK), lambda t, c, keys: (t, 0, c)),
        ],
        out_specs=[
            pl.BlockSpec((1, _B, 1), lambda t, c, keys: (t, 0, 0)),
            pl.BlockSpec((1, _B, 1), lambda t, c, keys: (t, 0, 0)),
        ],
        scratch_shapes=[
            pltpu.VMEM((_B, _TW), jnp.float32),
            pltpu.VMEM((_B, _TW), jnp.int32),
            pltpu.VMEM((_B, _TW), jnp.float32),
            pltpu.VMEM((_B, _TW), jnp.float32),
        ],
    )
    gen, logp = pl.pallas_call(
        _sample_kernel,
        grid_spec=grid_spec,
        out_shape=[
            jax.ShapeDtypeStruct((_L, _B, 1), jnp.int32),
            jax.ShapeDtypeStruct((_L, _B, 1), jnp.float32),
        ],
        compiler_params=pltpu.CompilerParams(
            dimension_semantics=("arbitrary", "arbitrary"),
        ),
    )(keys_flat, inp_lb, msk_lb, gen_logits)
    return gen, logp


def kernel(input_tensor, mask_tensor, gen_logits):
    L, B, V = gen_logits.shape
    assert (L, B, V) == (_L, _B, _V)

    sample_key = jax.random.key(42)
    keys = jax.vmap(
        lambda t: jax.random.key_data(jax.random.fold_in(sample_key, t))
    )(jnp.arange(L, dtype=jnp.uint32))
    keys_flat = keys.reshape(-1).astype(jnp.int32)

    inp_lb = input_tensor.T.reshape(L, B, 1)
    msk_lb = mask_tensor.T.reshape(L, B, 1)

    gen, logp = _run(keys_flat, inp_lb, msk_lb, gen_logits)
    generated = gen.reshape(L, B).T
    log_probs = logp.reshape(L, B).T
    return generated, log_probs


# speedup vs baseline: 1.3431x; 1.0115x over previous
import functools

import jax
import jax.numpy as jnp
import numpy as np
from jax import lax
from jax.experimental import pallas as pl
from jax.experimental.pallas import tpu as pltpu

_L = 32
_B = 32
_V = 100000
_CHUNK = 25600
_NCHUNK = (_V + _CHUNK - 1) // _CHUNK
_TW = 256
_NT = _CHUNK // _TW
_LAST_FULL = (_V - (_NCHUNK - 1) * _CHUNK) // _TW

_NEG_INF = np.float32(-np.inf)
_TINY = np.float32(np.finfo(np.float32).tiny)
_INT_MAX = np.int32(2**31 - 1)


def _rotl(x, r):
    return (x << jnp.uint32(r)) | (x >> jnp.uint32(32 - r))


def _threefry_bits(base_j, off_plus_ks1, k0, k1):
    ks0, ks1 = k0, k1
    ks2 = jnp.uint32(0x1BD11BDA) ^ k0 ^ k1
    ks = (ks0, ks1, ks2)
    rots = ((13, 15, 26, 6), (17, 29, 16, 24))
    x1 = base_j + off_plus_ks1
    x0 = x1 + ks0
    first = True
    for i in range(5):
        for r in rots[i % 2]:
            if first:
                first = False
            else:
                x0 = x0 + x1
            x1 = _rotl(x1, r)
            x1 = x0 ^ x1
        x0 = x0 + ks[(i + 1) % 3]
        x1 = x1 + ks[(i + 2) % 3] + jnp.uint32(i + 1)
    return x0 ^ x1


def _neg_gumbel_from_bits(bits):
    fb = (bits >> jnp.uint32(9)) | jnp.uint32(0x3F800000)
    fl = lax.bitcast_convert_type(fb, jnp.float32) - jnp.float32(1.0)
    u = fl + _TINY
    return jnp.log(-jnp.log(u))


def _sample_kernel(keys_ref, inp_ref, msk_ref, logits_ref, gen_ref, logp_ref,
                   bv_ref, bc_ref, sa_ref, ia_ref):
    t = pl.program_id(0)
    c = pl.program_id(1)

    @pl.when(c == 0)
    def _init():
        bv_ref[...] = jnp.full((_B, _TW), _NEG_INF, jnp.float32)
        bc_ref[...] = jnp.zeros((_B, _TW), jnp.int32)
        sa_ref[...] = jnp.zeros((_B, _TW), jnp.float32)
        ia_ref[...] = jnp.full((_B, _TW), _NEG_INF, jnp.float32)

    k0 = keys_ref[2 * t].astype(jnp.uint32)
    k1 = keys_ref[2 * t + 1].astype(jnp.uint32)
    inp = inp_ref[0]

    lane = lax.broadcasted_iota(jnp.int32, (_B, _TW), 1)
    row = lax.broadcasted_iota(jnp.uint32, (_B, _TW), 0)
    base_j = row * jnp.uint32(_V) + lane.astype(jnp.uint32)

    last = c == _NCHUNK - 1
    ngroups = jnp.where(last, _LAST_FULL // 10, _NT // 10)

    def tile(off, carry, masked):
        bv, bc, sa, ia = carry
        lo = logits_ref[0, :, pl.ds(off, _TW)]
        goff = c * _CHUNK + off
        ng = _neg_gumbel_from_bits(
            _threefry_bits(base_j, goff.astype(jnp.uint32) + k1, k0, k1))
        col = lane + goff
        if masked:
            valid = col < _V
            val = jnp.where(valid, lo - ng, _NEG_INF)
            sa = sa + jnp.where(valid, jnp.exp(lo), jnp.float32(0.0))
        else:
            val = lo - ng
            sa = sa + jnp.exp(lo)
        upd = val > bv
        bv = jnp.maximum(bv, val)
        bc = jnp.where(upd, col, bc)
        ia = jnp.maximum(ia, jnp.where(col == inp, lo, _NEG_INF))
        return bv, bc, sa, ia

    def group_body(i, carry):
        for u in range(10):
            carry = tile((10 * i + u) * _TW, carry, masked=False)
        return carry

    def tail_body(_, carry):
        return tile(_LAST_FULL * _TW, carry, masked=True)

    carry = (bv_ref[...], bc_ref[...], sa_ref[...], ia_ref[...])
    carry = lax.fori_loop(0, ngroups, group_body, carry)
    carry = lax.fori_loop(0, jnp.where(last, 1, 0), tail_body, carry)
    bv_ref[...], bc_ref[...], sa_ref[...], ia_ref[...] = carry

    @pl.when(last)
    def _finalize():
        bv, bc, sa, ia = bv_ref[...], bc_ref[...], sa_ref[...], ia_ref[...]
        cmax = jnp.max(bv, axis=1, keepdims=True)
        eq = bv == cmax
        samp = jnp.min(jnp.where(eq, bc, _INT_MAX), axis=1, keepdims=True)
        csum = jnp.sum(sa, axis=1, keepdims=True)
        cinp = jnp.max(ia, axis=1, keepdims=True)

        row1 = lax.broadcasted_iota(jnp.uint32, (_B, 1), 0)
        ngb = _neg_gumbel_from_bits(
            _threefry_bits(row1 * jnp.uint32(_V), samp.astype(jnp.uint32) + k1,
                           k0, k1))
        logit_b = cmax + ngb

        msk = msk_ref[0] != 0
        gen_ref[0] = jnp.where(msk, samp, inp)
        logp_ref[0] = jnp.where(msk, logit_b, cinp) - jnp.log(csum)


@jax.jit
def _run(keys_flat, inp_lb, msk_lb, gen_logits):
    grid_spec = pltpu.PrefetchScalarGridSpec(
        num_scalar_prefetch=1,
        grid=(_L, _NCHUNK),
        in_specs=[
            pl.BlockSpec((1, _B, 1), lambda t, c, keys: (t, 0, 0)),
            pl.BlockSpec((1, _B, 1), lambda t, c, keys: (t, 0, 0)),
            pl.BlockSpec((1, _B, _CHUNK), lambda t, c, keys: (t, 0, c)),
        ],
        out_specs=[
            pl.BlockSpec((1, _B, 1), lambda t, c, keys: (t, 0, 0)),
            pl.BlockSpec((1, _B, 1), lambda t, c, keys: (t, 0, 0)),
        ],
        scratch_shapes=[
            pltpu.VMEM((_B, _TW), jnp.float32),
            pltpu.VMEM((_B, _TW), jnp.int32),
            pltpu.VMEM((_B, _TW), jnp.float32),
            pltpu.VMEM((_B, _TW), jnp.float32),
        ],
    )
    gen, logp = pl.pallas_call(
        _sample_kernel,
        grid_spec=grid_spec,
        out_shape=[
            jax.ShapeDtypeStruct((_L, _B, 1), jnp.int32),
            jax.ShapeDtypeStruct((_L, _B, 1), jnp.float32),
        ],
        compiler_params=pltpu.CompilerParams(
            dimension_semantics=("arbitrary", "arbitrary"),
        ),
    )(keys_flat, inp_lb, msk_lb, gen_logits)
    return gen, logp


def kernel(input_tensor, mask_tensor, gen_logits):
    L, B, V = gen_logits.shape
    assert (L, B, V) == (_L, _B, _V)

    sample_key = jax.random.key(42)
    keys = jax.vmap(
        lambda t: jax.random.key_data(jax.random.fold_in(sample_key, t))
    )(jnp.arange(L, dtype=jnp.uint32))
    keys_flat = keys.reshape(-1).astype(jnp.int32)

    inp_lb = input_tensor.T.reshape(L, B, 1)
    msk_lb = mask_tensor.T.reshape(L, B, 1)

    gen, logp = _run(keys_flat, inp_lb, msk_lb, gen_logits)
    generated = gen.reshape(L, B).T
    log_probs = logp.reshape(L, B).T
    return generated, log_probs
